# Initial kernel scaffold; baseline (speedup 1.0000x reference)
import jax
import jax.numpy as jnp
from jax.experimental import pallas as pl


def kernel(rays_o, rays_d, z_vals, weights):
    raise NotImplementedError("write your pallas kernel here")


# speedup vs baseline: 2664.2480x; 2664.2480x over previous
import functools

import jax
import jax.numpy as jnp
from jax import lax
from jax.experimental import pallas as pl
from jax.experimental.pallas import tpu as pltpu
from jax.experimental.pallas import tpu_sc as plsc

N_RAYS = 65536
NS = 64
NF = 128
NT = NS + NF
EPS = 1e-5
NW = 32
RPW = N_RAYS // NW
CH = 128
NBLK = RPW // CH
MB = 144
MT = NT // 8
NRT = N_RAYS // 128

f32 = jnp.float32
i32 = jnp.int32


def _ceili(x):
    t = x.astype(i32)
    return t + jnp.where(t.astype(f32) < x, 1, 0)


def _sampler_body(z_hbm, w_hbm, u_hbm, zs_hbm,
                  zb, wb, uv, cdf, bins, slope, pos, cnt,
                  bkA, bkB, obuf, sem):
    nc = 2
    wid = lax.axis_index("s") * nc + lax.axis_index("c")
    pltpu.sync_copy(u_hbm, uv)
    iota = lax.iota(i32, 16)
    neg1 = jnp.full((16,), -1, i32)

    for c in range(MB // 16):
        bkA[pl.ds(c * 16, 16)] = neg1
        bkB[pl.ds(c * 16, 16)] = neg1

    def block_body(b, carry):
        rt = wid * NBLK + b
        r0 = rt * CH
        pltpu.sync_copy(z_hbm.at[pl.ds(r0, CH)], zb)
        pltpu.sync_copy(w_hbm.at[pl.ds(r0, CH)], wb)

        def ray_body(rr, carry2):
            rsp = jnp.full((16,), rr, i32)

            csums = []
            tot = 0.0
            for c in range(4):
                w = wb[rr, pl.ds(c * 16, 16)] + EPS
                if c == 0:
                    w = jnp.where(iota == 0, 0.0, w)
                if c == 3:
                    w = jnp.where(iota == 15, 0.0, w)
                cs = plsc.cumsum(w) + tot
                tot = cs[15]
                csums.append(cs)
            inv = 1.0 / jnp.full((16,), tot, f32)
            cdfs = []
            for c in range(4):
                cc = csums[c] * inv
                cdf[pl.ds(c * 16, 16)] = cc
                cdfs.append(cc)

            zcs, bcs, pcs, ip1s = [], [], [], []
            for c in range(4):
                ioc = iota + (c * 16)
                ip1 = jnp.minimum(ioc + 1, NS - 1)
                ip1s.append(ip1)
                zc = zb[rr, pl.ds(c * 16, 16)]
                znx = plsc.load_gather(zb, [rsp, ip1])
                bc = 0.5 * (zc + znx)
                bins[pl.ds(c * 16, 16)] = bc
                zcs.append(zc)
                bcs.append(bc)
            for c in range(4):
                cnx = plsc.load_gather(cdf, [ip1s[c]])
                bnx = plsc.load_gather(bins, [ip1s[c]])
                den = cnx - cdfs[c]
                sl = (bnx - bcs[c]) / jnp.where(den < 1e-5, 1.0, den)
                if c == 3:
                    sl = jnp.where(iota >= 14, 0.0, sl)
                slope[pl.ds(c * 16, 16)] = sl
                pc = jnp.minimum(_ceili(127.0 * cdfs[c]), NF)
                if c == 3:
                    pc = jnp.where(iota == 15, 999, pc)
                pos[pl.ds(c * 16, 16)] = pc
                pcs.append(pc)
            for c in range(4):
                pnx = plsc.load_gather(pos, [ip1s[c]])
                mk = pcs[c] < pnx
                plsc.store_scatter(bkA, [jnp.minimum(pcs[c], MB - 4)],
                                   iota + (c * 16), mask=mk)

            cnts = []
            cm = -1
            for c in range(4):
                im1 = jnp.maximum(iota + (c * 16 - 1), 0)
                b_im1 = plsc.load_gather(bins, [im1])
                c_im1 = plsc.load_gather(cdf, [im1])
                tden = bcs[c] - b_im1
                bad = tden <= 0.0
                t = (zcs[c] - b_im1) / jnp.where(bad, 1.0, tden)
                t = jnp.where(bad, 0.0, t)
                fv = c_im1 + t * (cdfs[c] - c_im1)
                cc = jnp.clip(_ceili(127.0 * fv), 0, NF)
                if c == 0:
                    cc = jnp.where(iota == 0, 0, cc)
                if c == 3:
                    cc = jnp.where(iota == 15, NF, cc)
                cc = jnp.maximum(plsc.cummax(cc), cm)
                cm = cc[15]
                cnt[pl.ds(c * 16, 16)] = cc
                cnts.append(cc)
            for c in range(4):
                cnx = plsc.load_gather(cnt, [ip1s[c]])
                mk = cnts[c] < cnx
                if c == 3:
                    mk = jnp.logical_or(mk, iota == 15)
                plsc.store_scatter(bkB, [jnp.minimum(cnts[c], MB - 4)],
                                   iota + (c * 16), mask=mk)
                rank = iota + (c * 16) + cnts[c]
                plsc.store_scatter(obuf, [rank, rsp], zcs[c])

            blc = -1
            mmc = -1
            for c in range(8):
                bl = jnp.maximum(plsc.cummax(bkA[pl.ds(c * 16, 16)]), blc)
                blc = bl[15]
                bkA[pl.ds(c * 16, 16)] = neg1
                mm = jnp.maximum(plsc.cummax(bkB[pl.ds(c * 16, 16)]), mmc)
                mmc = mm[15]
                bkB[pl.ds(c * 16, 16)] = neg1
                bg = plsc.load_gather(bins, [bl])
                cg = plsc.load_gather(cdf, [bl])
                sg = plsc.load_gather(slope, [bl])
                uc = uv[pl.ds(c * 16, 16)]
                s = bg + (uc - cg) * sg
                rank = (iota + (c * 16 + 1)) + mm
                plsc.store_scatter(obuf, [rank, rsp], s)
            bkA[pl.ds(NF, 16)] = neg1
            bkB[pl.ds(NF, 16)] = neg1
            return carry2

        lax.fori_loop(0, CH, ray_body, 0)
        handles = []
        for mt in range(MT):
            handles.append(pltpu.async_copy(
                obuf.at[pl.ds(mt * 8, 8), :], zs_hbm.at[mt, rt], sem))
        for h in handles:
            h.wait()
        return carry

    lax.fori_loop(0, NBLK, block_body, 0)


_sampler = functools.partial(
    pl.kernel,
    out_type=jax.ShapeDtypeStruct((MT, NRT, 8, 128), f32),
    mesh=plsc.VectorSubcoreMesh(core_axis_name="c", subcore_axis_name="s"),
    compiler_params=pltpu.CompilerParams(needs_layout_passes=False),
    scratch_types=[
        pltpu.VMEM((CH, NS), f32),
        pltpu.VMEM((CH, NS), f32),
        pltpu.VMEM((NF,), f32),
        pltpu.VMEM((NS,), f32),
        pltpu.VMEM((NS,), f32),
        pltpu.VMEM((NS,), f32),
        pltpu.VMEM((NS,), i32),
        pltpu.VMEM((NS,), i32),
        pltpu.VMEM((MB,), i32),
        pltpu.VMEM((MB,), i32),
        pltpu.VMEM((NT, CH), f32),
        pltpu.SemaphoreType.DMA,
    ],
)(_sampler_body)


BR = 4


def _expand_body(z_ref, o_ref, d_ref, out_ref):
    z = z_ref[...]
    for k in range(3):
        ok = o_ref[:, k, :]
        dk = d_ref[:, k, :]
        out_ref[k] = ok[None, :, None, :] + dk[None, :, None, :] * z


_expand = pl.pallas_call(
    _expand_body,
    grid=(NRT // BR,),
    in_specs=[
        pl.BlockSpec((MT, BR, 8, 128), lambda i: (0, i, 0, 0)),
        pl.BlockSpec((BR, 3, 128), lambda i: (i, 0, 0)),
        pl.BlockSpec((BR, 3, 128), lambda i: (i, 0, 0)),
    ],
    out_specs=pl.BlockSpec((3, MT, BR, 8, 128), lambda i: (0, 0, i, 0, 0)),
    out_shape=jax.ShapeDtypeStruct((3, MT, NRT, 8, 128), f32),
)


def kernel(rays_o, rays_d, z_vals, weights):
    u = jnp.linspace(0.0, 1.0, NF, dtype=f32)
    zs4 = _sampler(z_vals, weights, u)
    oT = rays_o.reshape(NRT, 128, 3).transpose(0, 2, 1)
    dT = rays_d.reshape(NRT, 128, 3).transpose(0, 2, 1)
    pts5 = _expand(zs4, oT, dT)
    pts = pts5.transpose(2, 4, 1, 3, 0).reshape(N_RAYS, NT, 3)
    zs = zs4.transpose(1, 3, 0, 2).reshape(N_RAYS, NT)
    return pts, zs
